# baseline (speedup 1.0000x reference)
import functools

import jax
import jax.numpy as jnp
from jax import lax
from jax.experimental import pallas as pl
from jax.experimental.pallas import tpu as pltpu
from jax.experimental.pallas import tpu_sc as plsc

_NUM_CORES = 2
_NUM_SUBCORES = 16
_NUM_WORKERS = _NUM_CORES * _NUM_SUBCORES
_L = 16
_CH = 256


@functools.partial(jax.jit, static_argnums=(2, 3, 4))
def _sc_gather_t(table4, idx, B, H, D):
    C = B // _NUM_WORKERS
    nhalf = H * (C // _CH)
    halves_per_h = C // _CH
    mesh = plsc.VectorSubcoreMesh(core_axis_name="c", subcore_axis_name="s")

    @functools.partial(
        pl.kernel,
        mesh=mesh,
        out_type=jax.ShapeDtypeStruct((H, D, B), jnp.float32),
        scratch_types=[
            pltpu.VMEM((_CH,), jnp.int32),
            pltpu.VMEM((_CH,), jnp.int32),
            pltpu.VMEM((_CH,), jnp.int32),
            pltpu.VMEM((_CH,), jnp.int32),
            pltpu.VMEM((_CH, 4 * D), jnp.float32),
            pltpu.VMEM((_CH, 4 * D), jnp.float32),
            pltpu.VMEM((D, _CH), jnp.float32),
            pltpu.VMEM((D, _CH), jnp.float32),
            pltpu.SemaphoreType.DMA,
            pltpu.SemaphoreType.DMA,
            pltpu.SemaphoreType.DMA,
            pltpu.SemaphoreType.DMA,
        ],
        compiler_params=pltpu.CompilerParams(
            use_tc_tiling_on_sc=True, needs_layout_passes=False
        ),
    )
    def k(t4_hbm, idx_hbm, out_hbm, i0, i1, q0, q1, r0, r1, t0, t1, g0, g1, w0, w1):
        idxv, qv, rows, trows = (i0, i1), (q0, q1), (r0, r1), (t0, t1)
        gsem, wsem = (g0, g1), (w0, w1)
        wid = lax.axis_index("s") * _NUM_CORES + lax.axis_index("c")
        base = wid * C
        iota = lax.iota(jnp.int32, _L)

        def stage(u, b):
            h = u // halves_per_h
            boff = (u % halves_per_h) * _CH
            pltpu.sync_copy(idx_hbm.at[pl.ds(h * B + base + boff, _CH)], idxv[b])

            def qbody(o, qc):
                v = idxv[b][pl.ds(o * _L, _L)]
                qv[b][pl.ds(o * _L, _L)] = lax.shift_right_logical(v, 2)
                return qc

            lax.fori_loop(0, _CH // _L, qbody, 0)
            pltpu.async_copy(t4_hbm.at[qv[b]], rows[b], gsem[b])

        for b in range(2):
            stage(b, b)

        def outer(g, carry):
            for b in range(2):
                u = g * 2 + b
                h = u // halves_per_h
                boff = (u % halves_per_h) * _CH
                pltpu.make_async_copy(
                    t4_hbm.at[qv[b]], rows[b], gsem[b]
                ).wait()

                @pl.when(g > 0)
                def _():
                    pltpu.make_async_copy(
                        trows[b],
                        out_hbm.at[h, :, pl.ds(base + boff, _CH)],
                        wsem[b],
                    ).wait()

                def tbody(o, tc):
                    jv = o * _L + iota
                    iv = idxv[b][pl.ds(o * _L, _L)]
                    cbase = lax.shift_left(
                        lax.bitwise_and(iv, jnp.int32(3)), jnp.int32(5)
                    )
                    for d in range(D):
                        v = plsc.load_gather(rows[b], [jv, cbase + d])
                        trows[b][d, pl.ds(o * _L, _L)] = v
                    return tc

                lax.fori_loop(0, _CH // _L, tbody, 0)

                pltpu.async_copy(
                    trows[b], out_hbm.at[h, :, pl.ds(base + boff, _CH)], wsem[b]
                )

                @pl.when(u + 2 < nhalf)
                def _():
                    stage(u + 2, b)

            return carry

        lax.fori_loop(0, nhalf // 2, outer, 0)

        for b in range(2):
            pltpu.make_async_copy(
                trows[b], out_hbm.at[0, :, pl.ds(base, _CH)], wsem[b]
            ).wait()

    return k(table4, idx)


def kernel(table, x):
    batch, hist = x.shape
    D = table.shape[1]
    table4 = table.reshape(-1, 4 * D)
    idx = x.T.reshape(-1).astype(jnp.int32)
    out_t = _sc_gather_t(table4, idx, batch, hist, D)
    return jnp.transpose(out_t, (2, 0, 1))
